# baseline (speedup 1.0000x reference)
import functools

import jax
import jax.numpy as jnp
from jax import lax
from jax.experimental import pallas as pl
from jax.experimental.pallas import tpu as pltpu
from jax.experimental.pallas import tpu_sc as plsc

N = 16384
K = 1000
B = 200
L = 16
CHUNK_OFFS = tuple(range(0, B - L, L)) + (B - L,)
NC = 2
NS = 16
NW = NC * NS
ROWS_PER_W = N // NW
R = 32
NB = ROWS_PER_W // R

_mesh = plsc.VectorSubcoreMesh(core_axis_name="c", subcore_axis_name="s")


@functools.partial(
    pl.kernel,
    mesh=_mesh,
    compiler_params=pltpu.CompilerParams(
        needs_layout_passes=False,
        use_tc_tiling_on_sc=False,
    ),
    out_type=jax.ShapeDtypeStruct((N * B,), jnp.float32),
    scratch_types=[
        pltpu.VMEM((R * K,), jnp.float32),
        pltpu.VMEM((R * B,), jnp.int32),
        pltpu.VMEM((R * B,), jnp.float32),
    ],
)
def _gather_rows(x_hbm, y_hbm, o_hbm, x_v, y_v, o_v):
    wid = lax.axis_index("s") * NC + lax.axis_index("c")
    base0 = wid * ROWS_PER_W

    def block(b, carry):
        base = base0 + b * R
        pltpu.sync_copy(x_hbm.at[pl.ds(base * K, R * K)], x_v)
        pltpu.sync_copy(y_hbm.at[pl.ds(base * B, R * B)], y_v)

        def row(i, carry2):
            rvec = jnp.full((L,), i * K, dtype=jnp.int32)
            for off in CHUNK_OFFS:
                idx = y_v[pl.ds(i * B + off, L)] + rvec
                vals = plsc.load_gather(x_v, [idx])
                o_v[pl.ds(i * B + off, L)] = vals
            return carry2

        lax.fori_loop(0, R, row, 0)
        pltpu.sync_copy(o_v, o_hbm.at[pl.ds(base * B, R * B)])
        return carry

    lax.fori_loop(0, NB, block, 0)


def kernel(x, y):
    out = _gather_rows(x.reshape(N * K), y.astype(jnp.int32).reshape(N * B))
    return out.reshape(N, B)


# speedup vs baseline: 1.1613x; 1.1613x over previous
import functools

import jax
import jax.numpy as jnp
from jax import lax
from jax.experimental import pallas as pl
from jax.experimental.pallas import tpu as pltpu
from jax.experimental.pallas import tpu_sc as plsc

N = 16384
K = 1000
B = 200
L = 16
CHUNK_OFFS = tuple(range(0, B - L, L)) + (B - L,)
NC = 2
NS = 16
NW = NC * NS
ROWS_PER_W = N // NW
R = 32
NB = ROWS_PER_W // R

_mesh = plsc.VectorSubcoreMesh(core_axis_name="c", subcore_axis_name="s")


@functools.partial(
    pl.kernel,
    mesh=_mesh,
    compiler_params=pltpu.CompilerParams(
        needs_layout_passes=False,
        use_tc_tiling_on_sc=False,
    ),
    out_type=jax.ShapeDtypeStruct((N * B,), jnp.float32),
    scratch_types=[
        pltpu.VMEM((R * K,), jnp.float32),
        pltpu.VMEM((R * K,), jnp.float32),
        pltpu.VMEM((R * B,), jnp.int32),
        pltpu.VMEM((R * B,), jnp.int32),
        pltpu.VMEM((R * B,), jnp.float32),
        pltpu.VMEM((R * B,), jnp.float32),
        pltpu.SemaphoreType.DMA,
        pltpu.SemaphoreType.DMA,
        pltpu.SemaphoreType.DMA,
        pltpu.SemaphoreType.DMA,
    ],
)
def _gather_rows(x_hbm, y_hbm, o_hbm,
                 x0, x1, y0, y1, o0, o1, si0, si1, so0, so1):
    wid = lax.axis_index("s") * NC + lax.axis_index("c")
    base0 = wid * ROWS_PER_W
    xs, ys, os_ = (x0, x1), (y0, y1), (o0, o1)
    sis, sos = (si0, si1), (so0, so1)

    def in_copies(g):
        base = base0 + g * R
        b = g % 2
        cx = pltpu.make_async_copy(
            x_hbm.at[pl.ds(base * K, R * K)], xs[b], sis[b])
        cy = pltpu.make_async_copy(
            y_hbm.at[pl.ds(base * B, R * B)], ys[b], sis[b])
        return cx, cy

    def out_copy(g):
        base = base0 + g * R
        b = g % 2
        return pltpu.make_async_copy(
            os_[b], o_hbm.at[pl.ds(base * B, R * B)], sos[b])

    cx, cy = in_copies(0)
    cx.start()
    cy.start()

    for g in range(NB):
        b = g % 2
        if g + 1 < NB:
            nx, ny = in_copies(g + 1)
            nx.start()
            ny.start()
        cx, cy = in_copies(g)
        cx.wait()
        cy.wait()
        if g >= 2:
            out_copy(g - 2).wait()
        x_v, y_v, o_v = xs[b], ys[b], os_[b]

        def row(i, carry2):
            rvec = jnp.full((L,), i * K, dtype=jnp.int32)
            for off in CHUNK_OFFS:
                idx = y_v[pl.ds(i * B + off, L)] + rvec
                vals = plsc.load_gather(x_v, [idx])
                o_v[pl.ds(i * B + off, L)] = vals
            return carry2

        lax.fori_loop(0, R, row, 0)
        out_copy(g).start()

    out_copy(NB - 2).wait()
    out_copy(NB - 1).wait()


def kernel(x, y):
    out = _gather_rows(x.reshape(N * K), y.astype(jnp.int32).reshape(N * B))
    return out.reshape(N, B)


# speedup vs baseline: 1.2818x; 1.1038x over previous
import functools

import jax
import jax.numpy as jnp
from jax import lax
from jax.experimental import pallas as pl
from jax.experimental.pallas import tpu as pltpu
from jax.experimental.pallas import tpu_sc as plsc

N = 16384
K = 1000
B = 200
L = 16
CHUNK_OFFS = tuple(range(0, B - L, L)) + (B - L,)
NC = 2
NS = 16
NW = NC * NS
ROWS_PER_W = N // NW
R = 32
NB = ROWS_PER_W // R

_mesh = plsc.VectorSubcoreMesh(core_axis_name="c", subcore_axis_name="s")


@functools.partial(
    pl.kernel,
    mesh=_mesh,
    compiler_params=pltpu.CompilerParams(
        needs_layout_passes=False,
        use_tc_tiling_on_sc=False,
    ),
    out_type=jax.ShapeDtypeStruct((N * B,), jnp.float32),
    scratch_types=[
        pltpu.VMEM((R * K,), jnp.float32),
        pltpu.VMEM((R * K,), jnp.float32),
        pltpu.VMEM((R * B,), jnp.int32),
        pltpu.VMEM((R * B,), jnp.int32),
        pltpu.VMEM((R * B,), jnp.float32),
        pltpu.VMEM((R * B,), jnp.float32),
        pltpu.SemaphoreType.DMA,
        pltpu.SemaphoreType.DMA,
        pltpu.SemaphoreType.DMA,
        pltpu.SemaphoreType.DMA,
    ],
)
def _gather_rows(x_hbm, y_hbm, o_hbm,
                 x0, x1, y0, y1, o0, o1, si0, si1, so0, so1):
    wid = lax.axis_index("s") * NC + lax.axis_index("c")
    base0 = wid * ROWS_PER_W
    xs, ys, os_ = (x0, x1), (y0, y1), (o0, o1)
    sis, sos = (si0, si1), (so0, so1)

    def in_copies(g):
        base = base0 + g * R
        b = g % 2
        cx = pltpu.make_async_copy(
            x_hbm.at[pl.ds(base * K, R * K)], xs[b], sis[b])
        cy = pltpu.make_async_copy(
            y_hbm.at[pl.ds(base * B, R * B)], ys[b], sis[b])
        return cx, cy

    def out_copy(g):
        base = base0 + g * R
        b = g % 2
        return pltpu.make_async_copy(
            os_[b], o_hbm.at[pl.ds(base * B, R * B)], sos[b])

    cx, cy = in_copies(0)
    cx.start()
    cy.start()

    for g in range(NB):
        b = g % 2
        if g + 1 < NB:
            nx, ny = in_copies(g + 1)
            nx.start()
            ny.start()
        cx, cy = in_copies(g)
        cx.wait()
        cy.wait()
        if g >= 2:
            out_copy(g - 2).wait()
        x_v, y_v, o_v = xs[b], ys[b], os_[b]

        @plsc.parallel_loop(0, R, 1, unroll=2)
        def row(i):
            rvec = jnp.full((L,), i * K, dtype=jnp.int32)
            for off in CHUNK_OFFS:
                idx = y_v[pl.ds(i * B + off, L)] + rvec
                vals = plsc.load_gather(x_v, [idx])
                o_v[pl.ds(i * B + off, L)] = vals

        out_copy(g).start()

    out_copy(NB - 2).wait()
    out_copy(NB - 1).wait()


def kernel(x, y):
    out = _gather_rows(x.reshape(N * K), y.astype(jnp.int32).reshape(N * B))
    return out.reshape(N, B)


# speedup vs baseline: 1.2834x; 1.0012x over previous
import functools

import jax
import jax.numpy as jnp
from jax import lax
from jax.experimental import pallas as pl
from jax.experimental.pallas import tpu as pltpu
from jax.experimental.pallas import tpu_sc as plsc

N = 16384
K = 1000
B = 200
L = 16
CHUNK_OFFS = tuple(range(0, B - L, L)) + (B - L,)
NC = 2
NS = 16
NW = NC * NS
ROWS_PER_W = N // NW
R = 32
NB = ROWS_PER_W // R

_mesh = plsc.VectorSubcoreMesh(core_axis_name="c", subcore_axis_name="s")


@functools.partial(
    pl.kernel,
    mesh=_mesh,
    compiler_params=pltpu.CompilerParams(
        needs_layout_passes=False,
        use_tc_tiling_on_sc=False,
    ),
    out_type=jax.ShapeDtypeStruct((N * B,), jnp.float32),
    scratch_types=[
        pltpu.VMEM((R * K,), jnp.float32),
        pltpu.VMEM((R * K,), jnp.float32),
        pltpu.VMEM((R * B,), jnp.int32),
        pltpu.VMEM((R * B,), jnp.int32),
        pltpu.VMEM((R * B,), jnp.float32),
        pltpu.VMEM((R * B,), jnp.float32),
        pltpu.SemaphoreType.DMA,
        pltpu.SemaphoreType.DMA,
        pltpu.SemaphoreType.DMA,
        pltpu.SemaphoreType.DMA,
    ],
)
def _gather_rows(x_hbm, y_hbm, o_hbm,
                 x0, x1, y0, y1, o0, o1, si0, si1, so0, so1):
    wid = lax.axis_index("s") * NC + lax.axis_index("c")
    base0 = wid * ROWS_PER_W
    xs, ys, os_ = (x0, x1), (y0, y1), (o0, o1)
    sis, sos = (si0, si1), (so0, so1)

    def in_copies(g):
        base = base0 + g * R
        b = g % 2
        cx = pltpu.make_async_copy(
            x_hbm.at[pl.ds(base * K, R * K)], xs[b], sis[b])
        cy = pltpu.make_async_copy(
            y_hbm.at[pl.ds(base * B, R * B)], ys[b], sis[b])
        return cx, cy

    def out_copy(g):
        base = base0 + g * R
        b = g % 2
        return pltpu.make_async_copy(
            os_[b], o_hbm.at[pl.ds(base * B, R * B)], sos[b])

    cx, cy = in_copies(0)
    cx.start()
    cy.start()

    for g in range(NB):
        b = g % 2
        if g + 1 < NB:
            nx, ny = in_copies(g + 1)
            nx.start()
            ny.start()
        cx, cy = in_copies(g)
        cx.wait()
        cy.wait()
        if g >= 2:
            out_copy(g - 2).wait()
        x_v, y_v, o_v = xs[b], ys[b], os_[b]

        @plsc.parallel_loop(0, R, 1, unroll=4)
        def row(i):
            rvec = jnp.full((L,), i * K, dtype=jnp.int32)
            for off in CHUNK_OFFS:
                idx = y_v[pl.ds(i * B + off, L)] + rvec
                vals = plsc.load_gather(x_v, [idx])
                o_v[pl.ds(i * B + off, L)] = vals

        out_copy(g).start()

    out_copy(NB - 2).wait()
    out_copy(NB - 1).wait()


def kernel(x, y):
    out = _gather_rows(x.reshape(N * K), y.astype(jnp.int32).reshape(N * B))
    return out.reshape(N, B)


# speedup vs baseline: 1.3226x; 1.0306x over previous
import functools

import jax
import jax.numpy as jnp
from jax import lax
from jax.experimental import pallas as pl
from jax.experimental.pallas import tpu as pltpu
from jax.experimental.pallas import tpu_sc as plsc

N = 16384
K = 1000
B = 200
L = 16
CHUNK_OFFS = tuple(range(0, B - L, L)) + (B - L,)
NC = 2
NS = 16
NW = NC * NS
ROWS_PER_W = N // NW
R = 32
NB = ROWS_PER_W // R

_mesh = plsc.VectorSubcoreMesh(core_axis_name="c", subcore_axis_name="s")


@functools.partial(
    pl.kernel,
    mesh=_mesh,
    compiler_params=pltpu.CompilerParams(
        needs_layout_passes=False,
        use_tc_tiling_on_sc=False,
    ),
    out_type=jax.ShapeDtypeStruct((N, B), jnp.float32),
    scratch_types=[
        pltpu.VMEM((R, K), jnp.float32),
        pltpu.VMEM((R, K), jnp.float32),
        pltpu.VMEM((R, B), jnp.int32),
        pltpu.VMEM((R, B), jnp.int32),
        pltpu.VMEM((R, B), jnp.float32),
        pltpu.VMEM((R, B), jnp.float32),
        pltpu.SemaphoreType.DMA,
        pltpu.SemaphoreType.DMA,
        pltpu.SemaphoreType.DMA,
        pltpu.SemaphoreType.DMA,
    ],
)
def _gather_rows(x_hbm, y_hbm, o_hbm,
                 x0, x1, y0, y1, o0, o1, si0, si1, so0, so1):
    wid = lax.axis_index("s") * NC + lax.axis_index("c")
    base0 = wid * ROWS_PER_W
    xs, ys, os_ = (x0, x1), (y0, y1), (o0, o1)
    sis, sos = (si0, si1), (so0, so1)

    def in_copies(g, b):
        base = base0 + g * R
        cx = pltpu.make_async_copy(
            x_hbm.at[pl.ds(base, R), :], xs[b], sis[b])
        cy = pltpu.make_async_copy(
            y_hbm.at[pl.ds(base, R), :], ys[b], sis[b])
        return cx, cy

    def out_copy(g, b):
        base = base0 + g * R
        return pltpu.make_async_copy(
            os_[b], o_hbm.at[pl.ds(base, R), :], sos[b])

    cx, cy = in_copies(0, 0)
    cx.start()
    cy.start()

    def pair(it, carry):
        for par in range(2):
            g = it * 2 + par

            @pl.when(g + 1 < NB)
            def _():
                nx, ny = in_copies(g + 1, 1 - par)
                nx.start()
                ny.start()

            cx, cy = in_copies(g, par)
            cx.wait()
            cy.wait()

            @pl.when(g >= 2)
            def _():
                out_copy(g - 2, par).wait()

            x_v, y_v, o_v = xs[par], ys[par], os_[par]

            @plsc.parallel_loop(0, R, 1, unroll=2)
            def row(i):
                rvec = jnp.full((L,), i, dtype=jnp.int32)
                for off in CHUNK_OFFS:
                    idx = y_v[i, pl.ds(off, L)]
                    vals = plsc.load_gather(x_v, [rvec, idx])
                    o_v[i, pl.ds(off, L)] = vals

            out_copy(g, par).start()
        return carry

    lax.fori_loop(0, NB // 2, pair, 0)
    out_copy(NB - 2, 0).wait()
    out_copy(NB - 1, 1).wait()


def kernel(x, y):
    return _gather_rows(x, y.astype(jnp.int32))


# speedup vs baseline: 2.2758x; 1.7207x over previous
import functools

import jax
import jax.numpy as jnp
from jax import lax
from jax.experimental import pallas as pl
from jax.experimental.pallas import tpu as pltpu
from jax.experimental.pallas import tpu_sc as plsc

N = 16384
K = 1000
B = 200
L = 16
CHUNK_OFFS = tuple(range(0, B - L, L)) + (B - L,)
NC = 2
NS = 16
NW = NC * NS
ROWS_PER_W = N // NW
R = 32
NB = ROWS_PER_W // R

_mesh = plsc.VectorSubcoreMesh(core_axis_name="c", subcore_axis_name="s")


@functools.partial(
    pl.kernel,
    mesh=_mesh,
    compiler_params=pltpu.CompilerParams(
        needs_layout_passes=False,
        use_tc_tiling_on_sc=True,
    ),
    out_type=jax.ShapeDtypeStruct((N, B), jnp.float32),
    scratch_types=[
        pltpu.VMEM((R, K), jnp.float32),
        pltpu.VMEM((R, K), jnp.float32),
        pltpu.VMEM((R, B), jnp.int32),
        pltpu.VMEM((R, B), jnp.int32),
        pltpu.VMEM((R, B), jnp.float32),
        pltpu.VMEM((R, B), jnp.float32),
        pltpu.SemaphoreType.DMA,
        pltpu.SemaphoreType.DMA,
        pltpu.SemaphoreType.DMA,
        pltpu.SemaphoreType.DMA,
    ],
)
def _gather_rows(x_hbm, y_hbm, o_hbm,
                 x0, x1, y0, y1, o0, o1, si0, si1, so0, so1):
    wid = lax.axis_index("s") * NC + lax.axis_index("c")
    base0 = wid * ROWS_PER_W
    xs, ys, os_ = (x0, x1), (y0, y1), (o0, o1)
    sis, sos = (si0, si1), (so0, so1)

    def in_copies(g, b):
        base = base0 + g * R
        cx = pltpu.make_async_copy(
            x_hbm.at[pl.ds(base, R), :], xs[b], sis[b])
        cy = pltpu.make_async_copy(
            y_hbm.at[pl.ds(base, R), :], ys[b], sis[b])
        return cx, cy

    def out_copy(g, b):
        base = base0 + g * R
        return pltpu.make_async_copy(
            os_[b], o_hbm.at[pl.ds(base, R), :], sos[b])

    cx, cy = in_copies(0, 0)
    cx.start()
    cy.start()

    def pair(it, carry):
        for par in range(2):
            g = it * 2 + par

            @pl.when(g + 1 < NB)
            def _():
                nx, ny = in_copies(g + 1, 1 - par)
                nx.start()
                ny.start()

            cx, cy = in_copies(g, par)
            cx.wait()
            cy.wait()

            @pl.when(g >= 2)
            def _():
                out_copy(g - 2, par).wait()

            x_v, y_v, o_v = xs[par], ys[par], os_[par]

            @plsc.parallel_loop(0, R, 1, unroll=2)
            def row(i):
                rvec = jnp.full((L,), i, dtype=jnp.int32)
                for off in CHUNK_OFFS:
                    idx = y_v[i, pl.ds(off, L)]
                    vals = plsc.load_gather(x_v, [rvec, idx])
                    o_v[i, pl.ds(off, L)] = vals

            out_copy(g, par).start()
        return carry

    lax.fori_loop(0, NB // 2, pair, 0)
    out_copy(NB - 2, 0).wait()
    out_copy(NB - 1, 1).wait()


def kernel(x, y):
    return _gather_rows(x, y.astype(jnp.int32))


# speedup vs baseline: 2.2784x; 1.0011x over previous
import functools

import jax
import jax.numpy as jnp
from jax import lax
from jax.experimental import pallas as pl
from jax.experimental.pallas import tpu as pltpu
from jax.experimental.pallas import tpu_sc as plsc

N = 16384
K = 1000
B = 200
L = 16
CHUNK_OFFS = tuple(range(0, B - L, L)) + (B - L,)
NC = 2
NS = 16
NW = NC * NS
ROWS_PER_W = N // NW
R = 32
NB = ROWS_PER_W // R

_mesh = plsc.VectorSubcoreMesh(core_axis_name="c", subcore_axis_name="s")


@functools.partial(
    pl.kernel,
    mesh=_mesh,
    compiler_params=pltpu.CompilerParams(
        needs_layout_passes=False,
        use_tc_tiling_on_sc=True,
        disable_bounds_checks=True,
        skip_device_barrier=True,
    ),
    out_type=jax.ShapeDtypeStruct((N, B), jnp.float32),
    scratch_types=[
        pltpu.VMEM((R, K), jnp.float32),
        pltpu.VMEM((R, K), jnp.float32),
        pltpu.VMEM((R, B), jnp.int32),
        pltpu.VMEM((R, B), jnp.int32),
        pltpu.VMEM((R, B), jnp.float32),
        pltpu.VMEM((R, B), jnp.float32),
        pltpu.SemaphoreType.DMA,
        pltpu.SemaphoreType.DMA,
        pltpu.SemaphoreType.DMA,
        pltpu.SemaphoreType.DMA,
    ],
)
def _gather_rows(x_hbm, y_hbm, o_hbm,
                 x0, x1, y0, y1, o0, o1, si0, si1, so0, so1):
    wid = lax.axis_index("s") * NC + lax.axis_index("c")
    base0 = wid * ROWS_PER_W
    xs, ys, os_ = (x0, x1), (y0, y1), (o0, o1)
    sis, sos = (si0, si1), (so0, so1)

    def in_copies(g, b):
        base = base0 + g * R
        cx = pltpu.make_async_copy(
            x_hbm.at[pl.ds(base, R), :], xs[b], sis[b])
        cy = pltpu.make_async_copy(
            y_hbm.at[pl.ds(base, R), :], ys[b], sis[b])
        return cx, cy

    def out_copy(g, b):
        base = base0 + g * R
        return pltpu.make_async_copy(
            os_[b], o_hbm.at[pl.ds(base, R), :], sos[b])

    cx, cy = in_copies(0, 0)
    cx.start()
    cy.start()

    def pair(it, carry):
        for par in range(2):
            g = it * 2 + par

            @pl.when(g + 1 < NB)
            def _():
                nx, ny = in_copies(g + 1, 1 - par)
                nx.start()
                ny.start()

            cx, cy = in_copies(g, par)
            cx.wait()
            cy.wait()

            @pl.when(g >= 2)
            def _():
                out_copy(g - 2, par).wait()

            x_v, y_v, o_v = xs[par], ys[par], os_[par]

            @plsc.parallel_loop(0, R, 1, unroll=2)
            def row(i):
                rvec = jnp.full((L,), i, dtype=jnp.int32)
                for off in CHUNK_OFFS:
                    idx = y_v[i, pl.ds(off, L)]
                    vals = plsc.load_gather(x_v, [rvec, idx])
                    o_v[i, pl.ds(off, L)] = vals

            out_copy(g, par).start()
        return carry

    lax.fori_loop(0, NB // 2, pair, 0)
    out_copy(NB - 2, 0).wait()
    out_copy(NB - 1, 1).wait()


def kernel(x, y):
    return _gather_rows(x, y.astype(jnp.int32))
